# baseline (speedup 1.0000x reference)
import functools

import jax
import jax.numpy as jnp
from jax import lax
from jax.experimental import pallas as pl
from jax.experimental.pallas import tpu as pltpu
from jax.experimental.pallas import tpu_sc as plsc

B = 4096
F = 512
D = 128
K = 64
REG = 0.01

_INT_MIN = -2147483648


_NC = 2
_NS = 16
_NW = _NC * _NS
_BPW = B // _NW


def _sc_gather_body(ub_hbm, pb_hbm, nb_hbm, uemb_hbm, iemb_hbm,
                    eu_out, ep_out, en_out,
                    idx_u, idx_p, idx_n, rows_u, rows_p, rows_n,
                    sem_u, sem_p, sem_n):
    wid = lax.axis_index("s") * _NC + lax.axis_index("c")
    base = wid * _BPW
    pltpu.sync_copy(ub_hbm.at[pl.ds(base, _BPW)], idx_u)
    pltpu.sync_copy(pb_hbm.at[pl.ds(base, _BPW)], idx_p)
    pltpu.sync_copy(nb_hbm.at[pl.ds(base, _BPW)], idx_n)
    cu = pltpu.async_copy(uemb_hbm.at[idx_u], rows_u, sem_u)
    cp = pltpu.async_copy(iemb_hbm.at[idx_p], rows_p, sem_p)
    cn = pltpu.async_copy(iemb_hbm.at[idx_n], rows_n, sem_n)
    cu.wait()
    pltpu.sync_copy(rows_u, eu_out.at[pl.ds(base, _BPW)])
    cp.wait()
    pltpu.sync_copy(rows_p, ep_out.at[pl.ds(base, _BPW)])
    cn.wait()
    pltpu.sync_copy(rows_n, en_out.at[pl.ds(base, _BPW)])


def _sc_gather(user_batch, pos_item_batch, neg_item_batch, user_emb, item_emb):
    out = jax.ShapeDtypeStruct((B, D), jnp.float32)
    f = pl.kernel(
        _sc_gather_body,
        out_type=(out, out, out),
        mesh=plsc.VectorSubcoreMesh(core_axis_name="c", subcore_axis_name="s"),
        scratch_types=(
            pltpu.VMEM((_BPW,), jnp.int32),
            pltpu.VMEM((_BPW,), jnp.int32),
            pltpu.VMEM((_BPW,), jnp.int32),
            pltpu.VMEM((_BPW, D), jnp.float32),
            pltpu.VMEM((_BPW, D), jnp.float32),
            pltpu.VMEM((_BPW, D), jnp.float32),
            pltpu.SemaphoreType.DMA,
            pltpu.SemaphoreType.DMA,
            pltpu.SemaphoreType.DMA,
        ),
    )
    return f(user_batch, pos_item_batch, neg_item_batch, user_emb, item_emb)


_BLK = 2048
_GRID = B // _BLK


def _tc_body(tau_ref, uf_ref, pf_ref, nf_ref, wu_ref, wi_ref, tri_ref,
             u_ref, p_ref, n_ref, sq_ref, acc_ref):
    i = pl.program_id(0)

    uf = uf_ref[...]
    bits = lax.bitcast_convert_type(uf, jnp.int32)
    k = jnp.where(bits >= 0, bits, jnp.int32(_INT_MIN) - bits)

    s = jnp.full((_BLK,), jnp.int32(_INT_MIN))
    for bit in range(31, -1, -1):
        cand = (jnp.zeros((_BLK,), jnp.int32) if bit == 31
                else s | jnp.int32(1 << bit))
        cnt = jnp.sum((k >= cand[:, None]).astype(jnp.float32), axis=1)
        s = jnp.where(cnt >= K, cand, s)
    t = s[:, None]

    gt = k > t
    eq = k == t
    cnt_gt = jnp.sum(gt.astype(jnp.float32), axis=1, keepdims=True)
    r = jnp.float32(K) - cnt_gt

    prefix = jnp.dot(eq.astype(jnp.bfloat16), tri_ref[...],
                     preferred_element_type=jnp.float32)
    mask = gt | (eq & (prefix <= r))

    mtau = jnp.where(mask, tau_ref[...], 0.0)
    uft = uf + mtau

    wu = wu_ref[...]
    wi = wi_ref[...]
    u_ref[...] = jnp.dot(uft, wu, preferred_element_type=jnp.float32)
    p_ref[...] = jnp.dot(pf_ref[...], wi, preferred_element_type=jnp.float32)
    n_ref[...] = jnp.dot(nf_ref[...], wi, preferred_element_type=jnp.float32)

    @pl.when(i == 0)
    def _init():
        acc_ref[0] = 0.0

    acc_ref[0] += jnp.sum(mtau * mtau)
    sq_ref[0] = acc_ref[0]


def _tc_stage1(tau, uf, pf, nf, wu, wi):
    tri = jnp.triu(jnp.ones((F, F), jnp.bfloat16))
    row_spec = pl.BlockSpec((_BLK, F), lambda i: (i, 0))
    emb_spec = pl.BlockSpec((_BLK, D), lambda i: (i, 0))
    w_spec = pl.BlockSpec((F, D), lambda i: (0, 0))
    tri_spec = pl.BlockSpec((F, F), lambda i: (0, 0))
    return pl.pallas_call(
        _tc_body,
        grid=(_GRID,),
        in_specs=[row_spec, row_spec, row_spec, row_spec, w_spec, w_spec,
                  tri_spec],
        out_specs=[
            emb_spec, emb_spec, emb_spec,
            pl.BlockSpec(memory_space=pltpu.SMEM),
        ],
        out_shape=[
            jax.ShapeDtypeStruct((B, D), jnp.float32),
            jax.ShapeDtypeStruct((B, D), jnp.float32),
            jax.ShapeDtypeStruct((B, D), jnp.float32),
            jax.ShapeDtypeStruct((1,), jnp.float32),
        ],
        scratch_shapes=[pltpu.SMEM((1,), jnp.float32)],
        compiler_params=pltpu.CompilerParams(
            dimension_semantics=("arbitrary",)),
    )(tau, uf, pf, nf, wu, wi, tri)


def _tc2_body(u_ref, p_ref, n_ref, eu_ref, ep_ref, en_ref, sq_ref,
              conf_ref, loss_ref, acc_ref):
    i = pl.program_id(0)
    u = u_ref[...] + eu_ref[...]
    pos = jnp.sum(u * (p_ref[...] + ep_ref[...]), axis=1)
    neg = jnp.sum(u * (n_ref[...] + en_ref[...]), axis=1)
    z = pos - neg
    conf = jnp.maximum(z, 0.0) + jnp.log1p(jnp.exp(-jnp.abs(z)))
    conf_ref[...] = conf

    @pl.when(i == 0)
    def _init():
        acc_ref[0] = 0.0

    acc_ref[0] += jnp.sum(conf)
    loss_ref[0] = acc_ref[0] + REG * jnp.sqrt(sq_ref[0])


def _tc_stage2(u, p, n, eu, ep, en, sq):
    emb_spec = pl.BlockSpec((_BLK, D), lambda i: (i, 0))
    return pl.pallas_call(
        _tc2_body,
        grid=(_GRID,),
        in_specs=[emb_spec] * 6 + [pl.BlockSpec(memory_space=pltpu.SMEM)],
        out_specs=[
            pl.BlockSpec((_BLK,), lambda i: (i,)),
            pl.BlockSpec(memory_space=pltpu.SMEM),
        ],
        out_shape=[
            jax.ShapeDtypeStruct((B,), jnp.float32),
            jax.ShapeDtypeStruct((1,), jnp.float32),
        ],
        scratch_shapes=[pltpu.SMEM((1,), jnp.float32)],
        compiler_params=pltpu.CompilerParams(
            dimension_semantics=("arbitrary",)),
    )(u, p, n, eu, ep, en, sq)


def kernel(tau, user_feature_batch, pos_item_feature_batch,
           neg_item_feature_batch, W_user, W_item, user_emb, item_emb,
           user_batch, pos_item_batch, neg_item_batch):
    eu, ep, en = _sc_gather(user_batch.astype(jnp.int32),
                            pos_item_batch.astype(jnp.int32),
                            neg_item_batch.astype(jnp.int32),
                            user_emb, item_emb)
    u, p, n, sq = _tc_stage1(tau, user_feature_batch, pos_item_feature_batch,
                             neg_item_feature_batch, W_user, W_item)
    conf, loss = _tc_stage2(u, p, n, eu, ep, en, sq)
    return loss[0], conf


# speedup vs baseline: 1.0543x; 1.0543x over previous
import functools

import jax
import jax.numpy as jnp
from jax import lax
from jax.experimental import pallas as pl
from jax.experimental.pallas import tpu as pltpu
from jax.experimental.pallas import tpu_sc as plsc

B = 4096
F = 512
D = 128
K = 64
REG = 0.01

_INT_MIN = -2147483648


_NC = 2
_NS = 16
_NW = _NC * _NS
_BPW = B // _NW


def _sc_gather_body(ub_hbm, pb_hbm, nb_hbm, uemb_hbm, iemb_hbm,
                    eu_out, ep_out, en_out,
                    idx_u, idx_p, idx_n, rows_u, rows_p, rows_n,
                    sem_u, sem_p, sem_n):
    wid = lax.axis_index("s") * _NC + lax.axis_index("c")
    base = wid * _BPW
    pltpu.sync_copy(ub_hbm.at[pl.ds(base, _BPW)], idx_u)
    pltpu.sync_copy(pb_hbm.at[pl.ds(base, _BPW)], idx_p)
    pltpu.sync_copy(nb_hbm.at[pl.ds(base, _BPW)], idx_n)
    cu = pltpu.async_copy(uemb_hbm.at[idx_u], rows_u, sem_u)
    cp = pltpu.async_copy(iemb_hbm.at[idx_p], rows_p, sem_p)
    cn = pltpu.async_copy(iemb_hbm.at[idx_n], rows_n, sem_n)
    cu.wait()
    pltpu.sync_copy(rows_u, eu_out.at[pl.ds(base, _BPW)])
    cp.wait()
    pltpu.sync_copy(rows_p, ep_out.at[pl.ds(base, _BPW)])
    cn.wait()
    pltpu.sync_copy(rows_n, en_out.at[pl.ds(base, _BPW)])


def _sc_gather(user_batch, pos_item_batch, neg_item_batch, user_emb, item_emb):
    out = jax.ShapeDtypeStruct((B, D), jnp.float32)
    f = pl.kernel(
        _sc_gather_body,
        out_type=(out, out, out),
        mesh=plsc.VectorSubcoreMesh(core_axis_name="c", subcore_axis_name="s"),
        scratch_types=(
            pltpu.VMEM((_BPW,), jnp.int32),
            pltpu.VMEM((_BPW,), jnp.int32),
            pltpu.VMEM((_BPW,), jnp.int32),
            pltpu.VMEM((_BPW, D), jnp.float32),
            pltpu.VMEM((_BPW, D), jnp.float32),
            pltpu.VMEM((_BPW, D), jnp.float32),
            pltpu.SemaphoreType.DMA,
            pltpu.SemaphoreType.DMA,
            pltpu.SemaphoreType.DMA,
        ),
    )
    return f(user_batch, pos_item_batch, neg_item_batch, user_emb, item_emb)


_BLK = 1024
_GRID = B // _BLK


def _tc_body(tau_ref, uf_ref, pf_ref, nf_ref, wu_ref, wi_ref, tri_ref,
             u_ref, p_ref, n_ref, sq_ref, acc_ref):
    i = pl.program_id(0)

    uf = uf_ref[...]
    bits = lax.bitcast_convert_type(uf, jnp.int32)
    k = jnp.where(bits >= 0, bits, jnp.int32(_INT_MIN) - bits)

    s = jnp.full((_BLK,), jnp.int32(_INT_MIN))
    for bit in range(31, -1, -1):
        cand = (jnp.zeros((_BLK,), jnp.int32) if bit == 31
                else s | jnp.int32(1 << bit))
        cnt = jnp.sum((k >= cand[:, None]).astype(jnp.float32), axis=1)
        s = jnp.where(cnt >= K, cand, s)
    t = s[:, None]

    gt = k > t
    eq = k == t
    cnt_gt = jnp.sum(gt.astype(jnp.float32), axis=1, keepdims=True)
    r = jnp.float32(K) - cnt_gt

    prefix = jnp.dot(eq.astype(jnp.bfloat16), tri_ref[...],
                     preferred_element_type=jnp.float32)
    mask = gt | (eq & (prefix <= r))

    mtau = jnp.where(mask, tau_ref[...], 0.0)
    uft = uf + mtau

    wu = wu_ref[...]
    wi = wi_ref[...]
    u_ref[...] = jnp.dot(uft, wu, preferred_element_type=jnp.float32)
    p_ref[...] = jnp.dot(pf_ref[...], wi, preferred_element_type=jnp.float32)
    n_ref[...] = jnp.dot(nf_ref[...], wi, preferred_element_type=jnp.float32)

    @pl.when(i == 0)
    def _init():
        acc_ref[0] = 0.0

    acc_ref[0] += jnp.sum(mtau * mtau)
    sq_ref[0] = acc_ref[0]


def _tc_stage1(tau, uf, pf, nf, wu, wi):
    tri = jnp.triu(jnp.ones((F, F), jnp.bfloat16))
    row_spec = pl.BlockSpec((_BLK, F), lambda i: (i, 0))
    emb_spec = pl.BlockSpec((_BLK, D), lambda i: (i, 0))
    w_spec = pl.BlockSpec((F, D), lambda i: (0, 0))
    tri_spec = pl.BlockSpec((F, F), lambda i: (0, 0))
    return pl.pallas_call(
        _tc_body,
        grid=(_GRID,),
        in_specs=[row_spec, row_spec, row_spec, row_spec, w_spec, w_spec,
                  tri_spec],
        out_specs=[
            emb_spec, emb_spec, emb_spec,
            pl.BlockSpec(memory_space=pltpu.SMEM),
        ],
        out_shape=[
            jax.ShapeDtypeStruct((B, D), jnp.float32),
            jax.ShapeDtypeStruct((B, D), jnp.float32),
            jax.ShapeDtypeStruct((B, D), jnp.float32),
            jax.ShapeDtypeStruct((1,), jnp.float32),
        ],
        scratch_shapes=[pltpu.SMEM((1,), jnp.float32)],
        compiler_params=pltpu.CompilerParams(
            dimension_semantics=("arbitrary",)),
    )(tau, uf, pf, nf, wu, wi, tri)


def _tc2_body(u_ref, p_ref, n_ref, eu_ref, ep_ref, en_ref, sq_ref,
              conf_ref, loss_ref, acc_ref):
    i = pl.program_id(0)
    u = u_ref[...] + eu_ref[...]
    pos = jnp.sum(u * (p_ref[...] + ep_ref[...]), axis=1)
    neg = jnp.sum(u * (n_ref[...] + en_ref[...]), axis=1)
    z = pos - neg
    conf = jnp.maximum(z, 0.0) + jnp.log1p(jnp.exp(-jnp.abs(z)))
    conf_ref[...] = conf

    @pl.when(i == 0)
    def _init():
        acc_ref[0] = 0.0

    acc_ref[0] += jnp.sum(conf)
    loss_ref[0] = acc_ref[0] + REG * jnp.sqrt(sq_ref[0])


def _tc_stage2(u, p, n, eu, ep, en, sq):
    emb_spec = pl.BlockSpec((_BLK, D), lambda i: (i, 0))
    return pl.pallas_call(
        _tc2_body,
        grid=(_GRID,),
        in_specs=[emb_spec] * 6 + [pl.BlockSpec(memory_space=pltpu.SMEM)],
        out_specs=[
            pl.BlockSpec((_BLK,), lambda i: (i,)),
            pl.BlockSpec(memory_space=pltpu.SMEM),
        ],
        out_shape=[
            jax.ShapeDtypeStruct((B,), jnp.float32),
            jax.ShapeDtypeStruct((1,), jnp.float32),
        ],
        scratch_shapes=[pltpu.SMEM((1,), jnp.float32)],
        compiler_params=pltpu.CompilerParams(
            dimension_semantics=("arbitrary",)),
    )(u, p, n, eu, ep, en, sq)


def kernel(tau, user_feature_batch, pos_item_feature_batch,
           neg_item_feature_batch, W_user, W_item, user_emb, item_emb,
           user_batch, pos_item_batch, neg_item_batch):
    eu, ep, en = _sc_gather(user_batch.astype(jnp.int32),
                            pos_item_batch.astype(jnp.int32),
                            neg_item_batch.astype(jnp.int32),
                            user_emb, item_emb)
    u, p, n, sq = _tc_stage1(tau, user_feature_batch, pos_item_feature_batch,
                             neg_item_feature_batch, W_user, W_item)
    conf, loss = _tc_stage2(u, p, n, eu, ep, en, sq)
    return loss[0], conf


# speedup vs baseline: 1.0548x; 1.0005x over previous
import jax
import jax.numpy as jnp
from jax import lax
from jax.experimental import pallas as pl
from jax.experimental.pallas import tpu as pltpu
from jax.experimental.pallas import tpu_sc as plsc

B = 4096
F = 512
D = 128
K = 64
REG = 0.01

_INT_MIN = -2147483648


_NC = 2
_NS = 16
_NW = _NC * _NS
_BPW = B // _NW


def _sc_gather_body(ub_hbm, pb_hbm, nb_hbm, uemb_hbm, iemb_hbm,
                    eu_out, ep_out, en_out,
                    idx_u, idx_p, idx_n, rows_u, rows_p, rows_n,
                    sem_u, sem_p, sem_n):
    wid = lax.axis_index("s") * _NC + lax.axis_index("c")
    base = wid * _BPW
    pltpu.sync_copy(ub_hbm.at[pl.ds(base, _BPW)], idx_u)
    pltpu.sync_copy(pb_hbm.at[pl.ds(base, _BPW)], idx_p)
    pltpu.sync_copy(nb_hbm.at[pl.ds(base, _BPW)], idx_n)
    cu = pltpu.async_copy(uemb_hbm.at[idx_u], rows_u, sem_u)
    cp = pltpu.async_copy(iemb_hbm.at[idx_p], rows_p, sem_p)
    cn = pltpu.async_copy(iemb_hbm.at[idx_n], rows_n, sem_n)
    cu.wait()
    pltpu.sync_copy(rows_u, eu_out.at[pl.ds(base, _BPW)])
    cp.wait()
    pltpu.sync_copy(rows_p, ep_out.at[pl.ds(base, _BPW)])
    cn.wait()
    pltpu.sync_copy(rows_n, en_out.at[pl.ds(base, _BPW)])


def _sc_gather(user_batch, pos_item_batch, neg_item_batch, user_emb, item_emb):
    out = jax.ShapeDtypeStruct((B, D), jnp.float32)
    f = pl.kernel(
        _sc_gather_body,
        out_type=(out, out, out),
        mesh=plsc.VectorSubcoreMesh(core_axis_name="c", subcore_axis_name="s"),
        scratch_types=(
            pltpu.VMEM((_BPW,), jnp.int32),
            pltpu.VMEM((_BPW,), jnp.int32),
            pltpu.VMEM((_BPW,), jnp.int32),
            pltpu.VMEM((_BPW, D), jnp.float32),
            pltpu.VMEM((_BPW, D), jnp.float32),
            pltpu.VMEM((_BPW, D), jnp.float32),
            pltpu.SemaphoreType.DMA,
            pltpu.SemaphoreType.DMA,
            pltpu.SemaphoreType.DMA,
        ),
    )
    return f(user_batch, pos_item_batch, neg_item_batch, user_emb, item_emb)


_BLK = 1024
_GRID = B // _BLK


def _tc_body(tau_ref, uf_ref, pf_ref, nf_ref, wu_ref, wi_ref, tri_ref,
             u_ref, p_ref, n_ref, sq_ref, acc_ref):
    i = pl.program_id(0)

    uf = uf_ref[...]
    bits = lax.bitcast_convert_type(uf, jnp.int32)
    k = jnp.where(bits >= 0, bits, jnp.int32(_INT_MIN) - bits)

    s = jnp.full((_BLK,), jnp.int32(_INT_MIN))
    for bit in range(31, -1, -1):
        cand = (jnp.zeros((_BLK,), jnp.int32) if bit == 31
                else s | jnp.int32(1 << bit))
        cnt = jnp.sum((k >= cand[:, None]).astype(jnp.float32), axis=1)
        s = jnp.where(cnt >= K, cand, s)
    t = s[:, None]

    gt = k > t
    eq = k == t
    cnt_gt = jnp.sum(gt.astype(jnp.float32), axis=1, keepdims=True)
    r = jnp.float32(K) - cnt_gt

    prefix = jnp.dot(eq.astype(jnp.bfloat16), tri_ref[...],
                     preferred_element_type=jnp.float32)
    mask = gt | (eq & (prefix <= r))

    mtau = jnp.where(mask, tau_ref[...], 0.0)
    uft = uf + mtau

    wu = wu_ref[...]
    wi = wi_ref[...]
    u_ref[...] = jnp.dot(uft, wu, preferred_element_type=jnp.float32)
    p_ref[...] = jnp.dot(pf_ref[...], wi, preferred_element_type=jnp.float32)
    n_ref[...] = jnp.dot(nf_ref[...], wi, preferred_element_type=jnp.float32)

    @pl.when(i == 0)
    def _init():
        acc_ref[0] = 0.0

    acc_ref[0] += jnp.sum(mtau * mtau)
    sq_ref[0] = acc_ref[0]


def _tc_stage1(tau, uf, pf, nf, wu, wi):
    tri = jnp.triu(jnp.ones((F, F), jnp.bfloat16))
    row_spec = pl.BlockSpec((_BLK, F), lambda i: (i, 0))
    emb_spec = pl.BlockSpec((_BLK, D), lambda i: (i, 0))
    w_spec = pl.BlockSpec((F, D), lambda i: (0, 0))
    tri_spec = pl.BlockSpec((F, F), lambda i: (0, 0))
    return pl.pallas_call(
        _tc_body,
        grid=(_GRID,),
        in_specs=[row_spec, row_spec, row_spec, row_spec, w_spec, w_spec,
                  tri_spec],
        out_specs=[
            emb_spec, emb_spec, emb_spec,
            pl.BlockSpec(memory_space=pltpu.SMEM),
        ],
        out_shape=[
            jax.ShapeDtypeStruct((B, D), jnp.float32),
            jax.ShapeDtypeStruct((B, D), jnp.float32),
            jax.ShapeDtypeStruct((B, D), jnp.float32),
            jax.ShapeDtypeStruct((1,), jnp.float32),
        ],
        scratch_shapes=[pltpu.SMEM((1,), jnp.float32)],
        compiler_params=pltpu.CompilerParams(
            dimension_semantics=("arbitrary",)),
    )(tau, uf, pf, nf, wu, wi, tri)


def _tc2_body(u_ref, p_ref, n_ref, eu_ref, ep_ref, en_ref, sq_ref,
              conf_ref, loss_ref, acc_ref):
    i = pl.program_id(0)
    u = u_ref[...] + eu_ref[...]
    pos = jnp.sum(u * (p_ref[...] + ep_ref[...]), axis=1)
    neg = jnp.sum(u * (n_ref[...] + en_ref[...]), axis=1)
    z = pos - neg
    conf = jnp.maximum(z, 0.0) + jnp.log1p(jnp.exp(-jnp.abs(z)))
    conf_ref[...] = conf

    @pl.when(i == 0)
    def _init():
        acc_ref[0] = 0.0

    acc_ref[0] += jnp.sum(conf)
    loss_ref[0] = acc_ref[0] + REG * jnp.sqrt(sq_ref[0])


def _tc_stage2(u, p, n, eu, ep, en, sq):
    emb_spec = pl.BlockSpec((_BLK, D), lambda i: (i, 0))
    return pl.pallas_call(
        _tc2_body,
        grid=(_GRID,),
        in_specs=[emb_spec] * 6 + [pl.BlockSpec(memory_space=pltpu.SMEM)],
        out_specs=[
            pl.BlockSpec((_BLK,), lambda i: (i,)),
            pl.BlockSpec(memory_space=pltpu.SMEM),
        ],
        out_shape=[
            jax.ShapeDtypeStruct((B,), jnp.float32),
            jax.ShapeDtypeStruct((1,), jnp.float32),
        ],
        scratch_shapes=[pltpu.SMEM((1,), jnp.float32)],
        compiler_params=pltpu.CompilerParams(
            dimension_semantics=("arbitrary",)),
    )(u, p, n, eu, ep, en, sq)


def kernel(tau, user_feature_batch, pos_item_feature_batch,
           neg_item_feature_batch, W_user, W_item, user_emb, item_emb,
           user_batch, pos_item_batch, neg_item_batch):
    eu, ep, en = _sc_gather(user_batch.astype(jnp.int32),
                            pos_item_batch.astype(jnp.int32),
                            neg_item_batch.astype(jnp.int32),
                            user_emb, item_emb)
    u, p, n, sq = _tc_stage1(tau, user_feature_batch, pos_item_feature_batch,
                             neg_item_feature_batch, W_user, W_item)
    conf, loss = _tc_stage2(u, p, n, eu, ep, en, sq)
    return loss[0], conf
